# baseline (speedup 1.0000x reference)
import jax
import jax.numpy as jnp
from jax import lax
from jax.experimental import pallas as pl
from jax.experimental.pallas import tpu as pltpu

_MAX_TILES = 4
_HIDDEN = 1280
_PATCHES = 1025
_NS = 32


def _body(ids_ref, gate_ref, ha_ref, hb_ref, table_ref, oa_ref, ob_ref):
    i = pl.program_id(0)
    g = jnp.tanh(gate_ref[0])
    ba = i // _MAX_TILES
    ta = i % _MAX_TILES
    rowa = table_ref[ids_ref[ba], ta]
    oa_ref[...] = ha_ref[...] + (rowa * g).reshape(1, 1, _HIDDEN)
    j = i + _NS // 2
    bb = j // _MAX_TILES
    tb = j % _MAX_TILES
    rowb = table_ref[ids_ref[bb], tb]
    ob_ref[...] = hb_ref[...] + (rowb * g).reshape(1, 1, _HIDDEN)


def kernel(hidden_state, aspect_ratio_ids, embedding_table, gate):
    batch = hidden_state.shape[0]
    ids = aspect_ratio_ids.astype(jnp.int32)
    table = embedding_table.reshape(-1, _MAX_TILES, 1, _HIDDEN)
    n_rows = table.shape[0]
    hid3 = hidden_state.reshape(_NS, _PATCHES, _HIDDEN)
    half = _NS // 2

    oa, ob = pl.pallas_call(
        _body,
        grid=(half,),
        in_specs=[
            pl.BlockSpec(memory_space=pltpu.SMEM),
            pl.BlockSpec(memory_space=pltpu.SMEM),
            pl.BlockSpec((1, _PATCHES, _HIDDEN), lambda i: (i, 0, 0)),
            pl.BlockSpec((1, _PATCHES, _HIDDEN), lambda i: (i + half, 0, 0)),
            pl.BlockSpec((n_rows, _MAX_TILES, 1, _HIDDEN),
                         lambda i: (0, 0, 0, 0)),
        ],
        out_specs=[
            pl.BlockSpec((1, _PATCHES, _HIDDEN), lambda i: (i, 0, 0)),
            pl.BlockSpec((1, _PATCHES, _HIDDEN), lambda i: (i, 0, 0)),
        ],
        out_shape=[
            jax.ShapeDtypeStruct((half, _PATCHES, _HIDDEN), jnp.float32),
            jax.ShapeDtypeStruct((half, _PATCHES, _HIDDEN), jnp.float32),
        ],
        compiler_params=pltpu.CompilerParams(
            dimension_semantics=("parallel",),
            vmem_limit_bytes=100 * 1024 * 1024,
        ),
    )(ids, gate, hid3, hid3, table)

    out = jnp.concatenate([oa, ob], axis=0).reshape(hidden_state.shape)
    return out


# speedup vs baseline: 3.5557x; 3.5557x over previous
import jax
import jax.numpy as jnp
from jax.experimental import pallas as pl
from jax.experimental.pallas import tpu as pltpu

_MAX_TILES = 4
_HIDDEN = 1280
_PATCHES = 1025


def _body(ids_ref, gate_ref, hid_ref, emb_ref, out_ref):
    g = jnp.tanh(gate_ref[0])
    out_ref[...] = hid_ref[...] + emb_ref[...] * g


def kernel(hidden_state, aspect_ratio_ids, embedding_table, gate):
    batch = hidden_state.shape[0]
    ids = aspect_ratio_ids.astype(jnp.int32)
    table = embedding_table.reshape(-1, _MAX_TILES, 1, _HIDDEN)
    grid = (batch, _MAX_TILES // 2)

    out = pl.pallas_call(
        _body,
        grid_spec=pltpu.PrefetchScalarGridSpec(
            num_scalar_prefetch=2,
            grid=grid,
            in_specs=[
                pl.BlockSpec(
                    (1, 2, _PATCHES, _HIDDEN),
                    lambda b, t, ids, gate: (b, t, 0, 0),
                ),
                pl.BlockSpec(
                    (1, 2, 1, _HIDDEN),
                    lambda b, t, ids, gate: (ids[b], t, 0, 0),
                ),
            ],
            out_specs=pl.BlockSpec(
                (1, 2, _PATCHES, _HIDDEN),
                lambda b, t, ids, gate: (b, t, 0, 0),
            ),
        ),
        out_shape=jax.ShapeDtypeStruct(hidden_state.shape, hidden_state.dtype),
        compiler_params=pltpu.CompilerParams(
            dimension_semantics=("parallel", "parallel"),
            vmem_limit_bytes=100 * 1024 * 1024,
        ),
    )(ids, gate, hidden_state, table)
    return out
